# baseline (speedup 1.0000x reference)
import functools

import jax
import jax.numpy as jnp
from jax.experimental import pallas as pl
from jax.experimental.pallas import tpu as pltpu

B = 16384
INPUT_DIM = 512
EMB_DIM = 128
N_CLUSTERS = 64
CHUNK = 2048
NCH = B // CHUNK
NBUF = 3


def _fused_body(x_hbm, pca_ref, cent_ref, out_ref, xv_ref, sem):
    def copy(k):
        return pltpu.make_async_copy(
            x_hbm.at[pl.ds(k * CHUNK, CHUNK), :],
            xv_ref.at[k % NBUF],
            sem.at[k % NBUF])

    for b in range(NBUF):
        copy(b).start()

    pe = pca_ref[...].astype(jnp.bfloat16)
    cen = cent_ref[...]
    cen_p = jnp.concatenate(
        [cen, jnp.zeros((128 - N_CLUSTERS, EMB_DIM), jnp.float32)], axis=0)
    c2 = jnp.sum(cen_p * cen_p, axis=1)[None, :]
    pad = jax.lax.broadcasted_iota(jnp.int32, (1, 128), 1) >= N_CLUSTERS
    c2 = jnp.where(pad, jnp.float32(3e38), c2)
    cen_bf = cen_p.astype(jnp.bfloat16)

    for k in range(NCH):
        copy(k).wait()
        xb = xv_ref[k % NBUF].astype(jnp.bfloat16)
        if k + NBUF < NCH:
            copy(k + NBUF).start()

        x_enc = jax.lax.dot_general(
            xb, pe, (((1,), (1,)), ((), ())),
            preferred_element_type=jnp.float32)
        cross = jax.lax.dot_general(
            x_enc.astype(jnp.bfloat16), cen_bf, (((1,), (1,)), ((), ())),
            preferred_element_type=jnp.float32)

        x2 = jnp.sum(x_enc * x_enc, axis=1, keepdims=True)
        m = jnp.min(c2 - 2.0 * cross, axis=1, keepdims=True)
        out_ref[pl.ds(k * CHUNK, CHUNK), :] = jnp.sqrt(
            jnp.maximum(x2 + m, 0.0))


@functools.partial(jax.jit, static_argnames=("interpret",))
def kernel(x, pca_components, centroids, interpret=False):
    return pl.pallas_call(
        _fused_body,
        in_specs=[
            pl.BlockSpec(memory_space=pl.ANY),
            pl.BlockSpec((EMB_DIM, INPUT_DIM), lambda: (0, 0)),
            pl.BlockSpec((N_CLUSTERS, EMB_DIM), lambda: (0, 0)),
        ],
        out_specs=pl.BlockSpec((B, 1), lambda: (0, 0)),
        out_shape=jax.ShapeDtypeStruct((B, 1), jnp.float32),
        scratch_shapes=[
            pltpu.VMEM((NBUF, CHUNK, INPUT_DIM), jnp.float32),
            pltpu.SemaphoreType.DMA((NBUF,)),
        ],
        interpret=interpret,
    )(x, pca_components, centroids).reshape(B)


# speedup vs baseline: 1.0058x; 1.0058x over previous
import functools

import jax
import jax.numpy as jnp
from jax.experimental import pallas as pl
from jax.experimental.pallas import tpu as pltpu

B = 16384
INPUT_DIM = 512
EMB_DIM = 128
N_CLUSTERS = 64
CHUNK = 4096
NCH = B // CHUNK
NBUF = NCH


def _fused_body(x_hbm, pca_ref, cent_ref, out_ref, xv_ref, sem):
    def copy(k):
        return pltpu.make_async_copy(
            x_hbm.at[pl.ds(k * CHUNK, CHUNK), :],
            xv_ref.at[k % NBUF],
            sem.at[k % NBUF])

    for b in range(NBUF):
        copy(b).start()

    pe = pca_ref[...].astype(jnp.bfloat16)
    cen = cent_ref[...]
    cen_p = jnp.concatenate(
        [cen, jnp.zeros((128 - N_CLUSTERS, EMB_DIM), jnp.float32)], axis=0)
    c2 = jnp.sum(cen_p * cen_p, axis=1)[None, :]
    pad = jax.lax.broadcasted_iota(jnp.int32, (1, 128), 1) >= N_CLUSTERS
    c2 = jnp.where(pad, jnp.float32(3e38), c2)
    cen_bf = cen_p.astype(jnp.bfloat16)

    for k in range(NCH):
        copy(k).wait()
        xb = xv_ref[k % NBUF].astype(jnp.bfloat16)

        x_enc = jax.lax.dot_general(
            xb, pe, (((1,), (1,)), ((), ())),
            preferred_element_type=jnp.float32)
        cross = jax.lax.dot_general(
            x_enc.astype(jnp.bfloat16), cen_bf, (((1,), (1,)), ((), ())),
            preferred_element_type=jnp.float32)

        x2 = jnp.sum(x_enc * x_enc, axis=1, keepdims=True)
        m = jnp.min(c2 - 2.0 * cross, axis=1, keepdims=True)
        out_ref[pl.ds(k * CHUNK, CHUNK), :] = jnp.sqrt(
            jnp.maximum(x2 + m, 0.0))


@functools.partial(jax.jit, static_argnames=("interpret",))
def kernel(x, pca_components, centroids, interpret=False):
    return pl.pallas_call(
        _fused_body,
        in_specs=[
            pl.BlockSpec(memory_space=pl.ANY),
            pl.BlockSpec((EMB_DIM, INPUT_DIM), lambda: (0, 0)),
            pl.BlockSpec((N_CLUSTERS, EMB_DIM), lambda: (0, 0)),
        ],
        out_specs=pl.BlockSpec((B, 1), lambda: (0, 0)),
        out_shape=jax.ShapeDtypeStruct((B, 1), jnp.float32),
        scratch_shapes=[
            pltpu.VMEM((NBUF, CHUNK, INPUT_DIM), jnp.float32),
            pltpu.SemaphoreType.DMA((NBUF,)),
        ],
        interpret=interpret,
    )(x, pca_components, centroids).reshape(B)


# speedup vs baseline: 1.0949x; 1.0886x over previous
import functools

import jax
import jax.numpy as jnp
from jax.experimental import pallas as pl
from jax.experimental.pallas import tpu as pltpu

B = 16384
INPUT_DIM = 512
EMB_DIM = 128
N_CLUSTERS = 64
BLOCK_ROWS = 4096


def _fused_body(x_ref, pca_ref, cent_ref, out_ref):
    xb = x_ref[...].astype(jnp.bfloat16)
    pe = pca_ref[...].astype(jnp.bfloat16)

    x_enc = jax.lax.dot_general(
        xb, pe, (((1,), (1,)), ((), ())),
        preferred_element_type=jnp.float32)

    cen = cent_ref[...]
    cen_p = jnp.concatenate(
        [cen, jnp.zeros((128 - N_CLUSTERS, EMB_DIM), jnp.float32)], axis=0)

    cross = jax.lax.dot_general(
        x_enc.astype(jnp.bfloat16), cen_p.astype(jnp.bfloat16),
        (((1,), (1,)), ((), ())),
        preferred_element_type=jnp.float32)

    x2 = jnp.sum(x_enc * x_enc, axis=1, keepdims=True)
    c2 = jnp.sum(cen_p * cen_p, axis=1)[None, :]
    pad = jax.lax.broadcasted_iota(jnp.int32, (1, 128), 1) >= N_CLUSTERS
    c2 = jnp.where(pad, jnp.float32(3e38), c2)
    m = jnp.min(c2 - 2.0 * cross, axis=1, keepdims=True)
    out_ref[...] = jnp.sqrt(jnp.maximum(x2 + m, 0.0))


@functools.partial(jax.jit, static_argnames=("interpret",))
def kernel(x, pca_components, centroids, interpret=False):
    grid = (B // BLOCK_ROWS,)
    return pl.pallas_call(
        _fused_body,
        grid=grid,
        in_specs=[
            pl.BlockSpec((BLOCK_ROWS, INPUT_DIM), lambda i: (i, 0)),
            pl.BlockSpec((EMB_DIM, INPUT_DIM), lambda i: (0, 0)),
            pl.BlockSpec((N_CLUSTERS, EMB_DIM), lambda i: (0, 0)),
        ],
        out_specs=pl.BlockSpec((BLOCK_ROWS, 1), lambda i: (i, 0)),
        out_shape=jax.ShapeDtypeStruct((B, 1), jnp.float32),
        interpret=interpret,
    )(x, pca_components, centroids).reshape(B)


# speedup vs baseline: 1.0967x; 1.0016x over previous
import functools

import jax
import jax.numpy as jnp
from jax.experimental import pallas as pl
from jax.experimental.pallas import tpu as pltpu

B = 16384
INPUT_DIM = 512
EMB_DIM = 128
N_CLUSTERS = 64
BLOCK_ROWS = 4096


def _fused_body(x_ref, pca_ref, cent_ref, out_ref):
    xb = x_ref[...]
    pe = pca_ref[...]

    x_enc = jax.lax.dot_general(
        xb, pe, (((1,), (1,)), ((), ())),
        preferred_element_type=jnp.float32)

    cen = cent_ref[...]
    cen_p = jnp.concatenate(
        [cen, jnp.zeros((128 - N_CLUSTERS, EMB_DIM), jnp.float32)], axis=0)

    cross = jax.lax.dot_general(
        x_enc, cen_p,
        (((1,), (1,)), ((), ())),
        preferred_element_type=jnp.float32)

    x2 = jnp.sum(x_enc * x_enc, axis=1, keepdims=True)
    c2 = jnp.sum(cen_p * cen_p, axis=1)[None, :]
    pad = jax.lax.broadcasted_iota(jnp.int32, (1, 128), 1) >= N_CLUSTERS
    c2 = jnp.where(pad, jnp.float32(3e38), c2)
    m = jnp.min(c2 - 2.0 * cross, axis=1, keepdims=True)
    out_ref[...] = jnp.sqrt(jnp.maximum(x2 + m, 0.0))


@functools.partial(jax.jit, static_argnames=("interpret",))
def kernel(x, pca_components, centroids, interpret=False):
    grid = (B // BLOCK_ROWS,)
    return pl.pallas_call(
        _fused_body,
        grid=grid,
        in_specs=[
            pl.BlockSpec((BLOCK_ROWS, INPUT_DIM), lambda i: (i, 0)),
            pl.BlockSpec((EMB_DIM, INPUT_DIM), lambda i: (0, 0)),
            pl.BlockSpec((N_CLUSTERS, EMB_DIM), lambda i: (0, 0)),
        ],
        out_specs=pl.BlockSpec((BLOCK_ROWS, 1), lambda i: (i, 0)),
        out_shape=jax.ShapeDtypeStruct((B, 1), jnp.float32),
        interpret=interpret,
    )(x, pca_components, centroids).reshape(B)


# speedup vs baseline: 1.6284x; 1.4848x over previous
import functools

import jax
import jax.numpy as jnp
from jax.experimental import pallas as pl

B = 16384
INPUT_DIM = 512
EMB_DIM = 128
N_CLUSTERS = 64
BLOCK_ROWS = 4096
NB = B // BLOCK_ROWS


def _fused_body(x_ref, pca_ref, cent_ref, out_ref):
    xb = x_ref[...]
    pe = pca_ref[...]

    x_enc_t = jax.lax.dot_general(
        pe, xb, (((1,), (1,)), ((), ())),
        preferred_element_type=jnp.float32)

    cen = cent_ref[...]
    cen_p = jnp.concatenate(
        [cen, jnp.zeros((128 - N_CLUSTERS, EMB_DIM), jnp.float32)], axis=0)

    cross_t = jax.lax.dot_general(
        cen_p, x_enc_t, (((1,), (0,)), ((), ())),
        preferred_element_type=jnp.float32)

    x_sq = x_enc_t * x_enc_t
    x2row = jax.lax.dot_general(
        jnp.ones((8, EMB_DIM), jnp.float32), x_sq, (((1,), (0,)), ((), ())),
        preferred_element_type=jnp.float32)[:1]

    c2 = jnp.sum(cen_p * cen_p, axis=1, keepdims=True)
    pad = jax.lax.broadcasted_iota(jnp.int32, (128, 1), 0) >= N_CLUSTERS
    c2 = jnp.where(pad, jnp.float32(3e38), c2)
    m = jnp.min(c2 - 2.0 * cross_t, axis=0, keepdims=True)
    out_ref[...] = jnp.sqrt(jnp.maximum(x2row + m, 0.0))[None]


@functools.partial(jax.jit, static_argnames=("interpret",))
def kernel(x, pca_components, centroids, interpret=False):
    return pl.pallas_call(
        _fused_body,
        grid=(NB,),
        in_specs=[
            pl.BlockSpec((BLOCK_ROWS, INPUT_DIM), lambda i: (i, 0)),
            pl.BlockSpec((EMB_DIM, INPUT_DIM), lambda i: (0, 0)),
            pl.BlockSpec((N_CLUSTERS, EMB_DIM), lambda i: (0, 0)),
        ],
        out_specs=pl.BlockSpec((1, 1, BLOCK_ROWS), lambda i: (i, 0, 0)),
        out_shape=jax.ShapeDtypeStruct((NB, 1, BLOCK_ROWS), jnp.float32),
        interpret=interpret,
    )(x, pca_components, centroids).reshape(B)
